# baseline (speedup 1.0000x reference)
import functools

import numpy as np
import jax
import jax.numpy as jnp
from jax import lax
from jax.experimental import pallas as pl
from jax.experimental.pallas import tpu as pltpu
from jax.experimental.pallas import tpu_sc as plsc

D_MODEL = 256
N_LEVELS = 4
N_HEADS = 8
N_POINTS = 4
SPATIAL = [(128, 128), (64, 64), (32, 32), (16, 16)]
LEN_IN = sum(h * w for h, w in SPATIAL)
NB = 2
LEN_Q = LEN_IN
D_HEAD = 32
NQ = NB * LEN_Q
NROWS = NQ * N_HEADS
HLP = N_HEADS * N_LEVELS * N_POINTS

_cols_l = np.tile(np.repeat(np.arange(N_LEVELS), N_POINTS), N_HEADS)
_cols_h = np.repeat(np.arange(N_HEADS), N_LEVELS * N_POINTS)
_Ws = np.array([w for h, w in SPATIAL], np.float32)
_Hs = np.array([h for h, w in SPATIAL], np.float32)
_starts = np.cumsum([0] + [h * w for h, w in SPATIAL])[:-1].astype(np.float32)
_META_NP = np.zeros((8, HLP), np.float32)
_META_NP[0] = _Ws[_cols_l]
_META_NP[1] = _Hs[_cols_l]
_META_NP[2] = _starts[_cols_l]
_META_NP[3] = _cols_h
_SEL_NP = (np.arange(N_LEVELS)[:, None] == _cols_l[None, :]).astype(np.float32)

_ILV = np.empty(32, np.int64)
_ILV[0::2] = np.arange(16)
_ILV[1::2] = np.arange(16) + 16
_VPERM = (np.repeat(np.arange(N_HEADS) * 32, 32) + np.tile(_ILV, N_HEADS))

BQ = 256
QT_PER_B = LEN_Q // BQ
BR = 512


def _mm_body(x_ref, w_ref, b_ref, o_ref):
    y = jnp.dot(x_ref[:], w_ref[:],
                preferred_element_type=jnp.float32) + b_ref[:]
    o_ref[:] = y.astype(o_ref.dtype)


def _matmul(x, W, b, br, out_dtype=jnp.float32):
    m, k = x.shape
    n = W.shape[1]
    return pl.pallas_call(
        _mm_body,
        grid=(m // br,),
        in_specs=[pl.BlockSpec((br, k), lambda i: (i, 0)),
                  pl.BlockSpec((k, n), lambda i: (0, 0)),
                  pl.BlockSpec((1, n), lambda i: (0, 0))],
        out_specs=pl.BlockSpec((br, n), lambda i: (i, 0)),
        out_shape=jax.ShapeDtypeStruct((m, n), out_dtype),
    )(x, W, b.reshape(1, n))


def _prep_body(q_ref, rp_ref, wso_ref, bso_ref, waw_ref, baw_ref,
               meta_ref, sel_ref, idx_ref, w_ref):
    q = q_ref[:]
    so = jnp.dot(q, wso_ref[:], preferred_element_type=jnp.float32) + bso_ref[:]
    sx = so[:, :HLP]
    sy = so[:, HLP:]
    aw = jnp.dot(q, waw_ref[:], preferred_element_type=jnp.float32) + baw_ref[:]
    parts = []
    for h in range(N_HEADS):
        s = aw[:, h * 16:(h + 1) * 16]
        m = jnp.max(s, axis=1, keepdims=True)
        e = jnp.exp(s - m)
        parts.append(e / jnp.sum(e, axis=1, keepdims=True))
    aws = jnp.concatenate(parts, axis=1)

    Wf = meta_ref[0:1, :]
    Hf = meta_ref[1:2, :]
    startf = meta_ref[2:3, :]
    hf = meta_ref[3:4, :]
    rpx = jnp.dot(rp_ref[:, 0:4], sel_ref[:], preferred_element_type=jnp.float32, precision=lax.Precision.HIGHEST)
    rpy = jnp.dot(rp_ref[:, 4:8], sel_ref[:], preferred_element_type=jnp.float32, precision=lax.Precision.HIGHEST)
    x = (rpx + sx / Wf) * Wf - 0.5
    y = (rpy + sy / Hf) * Hf - 0.5
    x0 = jnp.floor(x)
    y0 = jnp.floor(y)
    wx1 = x - x0
    wx0 = 1.0 - wx1
    wy1 = y - y0
    wy0 = 1.0 - wy1
    x1 = x0 + 1.0
    y1 = y0 + 1.0
    Wm1 = Wf - 1.0
    Hm1 = Hf - 1.0
    vx0 = ((x0 >= 0.0) & (x0 <= Wm1)).astype(jnp.float32)
    vx1 = ((x1 >= 0.0) & (x1 <= Wm1)).astype(jnp.float32)
    vy0 = ((y0 >= 0.0) & (y0 <= Hm1)).astype(jnp.float32)
    vy1 = ((y1 >= 0.0) & (y1 <= Hm1)).astype(jnp.float32)
    x0c = jnp.clip(x0, 0.0, Wm1)
    x1c = jnp.clip(x1, 0.0, Wm1)
    y0c = jnp.clip(y0, 0.0, Hm1)
    y1c = jnp.clip(y1, 0.0, Hm1)
    nb = (pl.program_id(0) // QT_PER_B).astype(jnp.float32) * float(LEN_IN * N_HEADS)

    def fidx(xc, yc):
        return (nb + (startf + yc * Wf + xc) * 8.0 + hf).astype(jnp.int32)

    wx0v = wx0 * vx0
    wx1v = wx1 * vx1
    wy0v = wy0 * vy0 * aws
    wy1v = wy1 * vy1 * aws
    idx_ref[:] = jnp.concatenate(
        [fidx(x0c, y0c), fidx(x1c, y0c), fidx(x0c, y1c), fidx(x1c, y1c)], axis=1)
    w_ref[:] = jnp.concatenate(
        [wx0v * wy0v, wx1v * wy0v, wx0v * wy1v, wx1v * wy1v], axis=1)


def _prep(q_flat, rp_flat, wso_p, bso_p, W_aw, b_aw, meta, sel):
    m = q_flat.shape[0]
    return pl.pallas_call(
        _prep_body,
        grid=(m // BQ,),
        in_specs=[pl.BlockSpec((BQ, D_MODEL), lambda i: (i, 0)),
                  pl.BlockSpec((BQ, 8), lambda i: (i, 0)),
                  pl.BlockSpec((D_MODEL, 2 * HLP), lambda i: (0, 0)),
                  pl.BlockSpec((1, 2 * HLP), lambda i: (0, 0)),
                  pl.BlockSpec((D_MODEL, HLP), lambda i: (0, 0)),
                  pl.BlockSpec((1, HLP), lambda i: (0, 0)),
                  pl.BlockSpec((8, HLP), lambda i: (0, 0)),
                  pl.BlockSpec((N_LEVELS, HLP), lambda i: (0, 0))],
        out_specs=[pl.BlockSpec((BQ, 4 * HLP), lambda i: (i, 0)),
                   pl.BlockSpec((BQ, 4 * HLP), lambda i: (i, 0))],
        out_shape=[jax.ShapeDtypeStruct((m, 4 * HLP), jnp.int32),
                   jax.ShapeDtypeStruct((m, 4 * HLP), jnp.float32)],
    )(q_flat, rp_flat, wso_p, bso_p.reshape(1, 2 * HLP), W_aw,
      b_aw.reshape(1, HLP), meta, sel)


NW = 32
G = 2
CH = G * 4


def _make_sc(nq):
    qpw = nq // NW
    nit = qpw // G

    def _sc_body(tab, idxh, wh, outh,
                 idx_v, w_v, rows_v, out_v, sem_g, sem_o):
        wid = lax.axis_index("s") * 2 + lax.axis_index("c")
        q0 = wid * qpw

        def stage(it, buf):
            qb = q0 + it * G
            pltpu.sync_copy(idxh.at[pl.ds(qb * 4, CH)], idx_v.at[buf])
            pltpu.sync_copy(wh.at[pl.ds(qb * 4, CH)], w_v.at[buf])
            for j in range(CH):
                pltpu.async_copy(tab.at[idx_v.at[buf, j]],
                                 rows_v.at[buf, pl.ds(j * 128, 128)],
                                 sem_g.at[buf])

        stage(0, 0)

        def it_body(it, carry):
            cur = lax.rem(it, 2)
            qb = q0 + it * G

            @pl.when(it + 1 < nit)
            def _():
                stage(it + 1, 1 - cur)

            for j in range(CH):
                pltpu.make_async_copy(tab.at[idx_v.at[cur, j]],
                                      rows_v.at[cur, pl.ds(j * 128, 128)],
                                      sem_g.at[cur]).wait()

            @pl.when(it >= 2)
            def _():
                pltpu.make_async_copy(out_v.at[cur],
                                      outh.at[pl.ds(qb, G)],
                                      sem_o.at[cur]).wait()

            for q in range(G):
                for h in range(N_HEADS):

                    def c_body(c, acc, q=q, h=h):
                        acc = list(acc)
                        wv = w_v[cur, q * 4 + c, pl.ds(h * 16, 16)]
                        ro = q * 512 + c * 128 + h * 16
                        for i in range(16):
                            wgt = wv[i]
                            k = (i & 3) * 2
                            acc[k] = acc[k] + wgt * rows_v[cur, ro + i, pl.ds(0, 16)]
                            acc[k + 1] = acc[k + 1] + wgt * rows_v[cur, ro + i, pl.ds(16, 16)]
                        return tuple(acc)

                    z = jnp.zeros((16,), jnp.float32)
                    acc = lax.fori_loop(0, 4, c_body, (z,) * 8)
                    a0 = (acc[0] + acc[2]) + (acc[4] + acc[6])
                    a1 = (acc[1] + acc[3]) + (acc[5] + acc[7])
                    out_v[cur, q, pl.ds(h * 32, 16)] = a0
                    out_v[cur, q, pl.ds(h * 32 + 16, 16)] = a1
            pltpu.async_copy(out_v.at[cur], outh.at[pl.ds(qb, G)],
                             sem_o.at[cur])
            return carry

        lax.fori_loop(0, nit, it_body, 0)
        for tail in (nit - 2, nit - 1):
            pltpu.make_async_copy(out_v.at[tail % 2],
                                  outh.at[pl.ds(q0 + tail * G, G)],
                                  sem_o.at[tail % 2]).wait()

    return functools.partial(
        pl.kernel,
        out_type=jax.ShapeDtypeStruct((nq, D_MODEL), jnp.float32),
        mesh=plsc.VectorSubcoreMesh(core_axis_name="c", subcore_axis_name="s"),
        scratch_types=[
            pltpu.VMEM((2, CH, 128), jnp.int32),
            pltpu.VMEM((2, CH, 128), jnp.float32),
            pltpu.VMEM((2, CH * 128, D_HEAD), jnp.float32),
            pltpu.VMEM((2, G, D_MODEL), jnp.float32),
            pltpu.SemaphoreType.DMA((2,)),
            pltpu.SemaphoreType.DMA((2,)),
        ],
        compiler_params=pltpu.CompilerParams(use_tc_tiling_on_sc=False),
    )(_sc_body)


_sc_gather_half = _make_sc(LEN_Q)


def kernel(query, reference_points, input_flatten, input_spatial_shapes,
           input_level_start_index, W_so, b_so, W_aw, b_aw, W_v, b_v,
           W_out, b_out):
    del input_spatial_shapes, input_level_start_index
    wso_p = W_so.reshape(D_MODEL, N_HEADS, N_LEVELS, N_POINTS, 2) \
        .transpose(0, 4, 1, 2, 3).reshape(D_MODEL, 2 * HLP)
    bso_p = b_so.reshape(N_HEADS, N_LEVELS, N_POINTS, 2) \
        .transpose(3, 0, 1, 2).reshape(2 * HLP)
    meta = jnp.asarray(_META_NP)
    sel = jnp.asarray(_SEL_NP)

    outs = []
    for n in range(NB):
        q_flat = query[n]
        rp_flat = reference_points[n].transpose(0, 2, 1).reshape(LEN_Q, 8)
        vtab = _matmul(input_flatten[n], W_v, b_v, BR)
        vtab = vtab.reshape(LEN_Q * N_HEADS, D_HEAD)
        idx, w = _prep(q_flat, rp_flat, wso_p, bso_p, W_aw, b_aw, meta, sel)
        out_sc = _sc_gather_half(vtab, idx.reshape(LEN_Q * 4, HLP),
                                 w.reshape(LEN_Q * 4, HLP))
        outs.append(_matmul(out_sc, W_out, b_out, BR))
    return jnp.stack(outs)


# speedup vs baseline: 1.0279x; 1.0279x over previous
import functools

import numpy as np
import jax
import jax.numpy as jnp
from jax import lax
from jax.experimental import pallas as pl
from jax.experimental.pallas import tpu as pltpu
from jax.experimental.pallas import tpu_sc as plsc

D_MODEL = 256
N_LEVELS = 4
N_HEADS = 8
N_POINTS = 4
SPATIAL = [(128, 128), (64, 64), (32, 32), (16, 16)]
LEN_IN = sum(h * w for h, w in SPATIAL)
NB = 2
LEN_Q = LEN_IN
D_HEAD = 32
NQ = NB * LEN_Q
NROWS = NQ * N_HEADS
HLP = N_HEADS * N_LEVELS * N_POINTS

_cols_l = np.tile(np.repeat(np.arange(N_LEVELS), N_POINTS), N_HEADS)
_cols_h = np.repeat(np.arange(N_HEADS), N_LEVELS * N_POINTS)
_Ws = np.array([w for h, w in SPATIAL], np.float32)
_Hs = np.array([h for h, w in SPATIAL], np.float32)
_starts = np.cumsum([0] + [h * w for h, w in SPATIAL])[:-1].astype(np.float32)
_META_NP = np.zeros((8, HLP), np.float32)
_META_NP[0] = _Ws[_cols_l]
_META_NP[1] = _Hs[_cols_l]
_META_NP[2] = _starts[_cols_l]
_META_NP[3] = _cols_h
_SEL_NP = (np.arange(N_LEVELS)[:, None] == _cols_l[None, :]).astype(np.float32)

_ILV = np.empty(32, np.int64)
_ILV[0::2] = np.arange(16)
_ILV[1::2] = np.arange(16) + 16
_VPERM = (np.repeat(np.arange(N_HEADS) * 32, 32) + np.tile(_ILV, N_HEADS))

BQ = 256
QT_PER_B = LEN_Q // BQ
BR = 512


def _mm_body(x_ref, w_ref, b_ref, o_ref):
    y = jnp.dot(x_ref[:], w_ref[:],
                preferred_element_type=jnp.float32) + b_ref[:]
    o_ref[:] = y.astype(o_ref.dtype)


def _matmul(x, W, b, br, out_dtype=jnp.float32):
    m, k = x.shape
    n = W.shape[1]
    return pl.pallas_call(
        _mm_body,
        grid=(m // br,),
        in_specs=[pl.BlockSpec((br, k), lambda i: (i, 0)),
                  pl.BlockSpec((k, n), lambda i: (0, 0)),
                  pl.BlockSpec((1, n), lambda i: (0, 0))],
        out_specs=pl.BlockSpec((br, n), lambda i: (i, 0)),
        out_shape=jax.ShapeDtypeStruct((m, n), out_dtype),
    )(x, W, b.reshape(1, n))


def _prep_body(q_ref, rp_ref, wso_ref, bso_ref, waw_ref, baw_ref,
               meta_ref, sel_ref, idx_ref, w_ref):
    q = q_ref[:]
    so = jnp.dot(q, wso_ref[:], preferred_element_type=jnp.float32) + bso_ref[:]
    sx = so[:, :HLP]
    sy = so[:, HLP:]
    aw = jnp.dot(q, waw_ref[:], preferred_element_type=jnp.float32) + baw_ref[:]
    parts = []
    for h in range(N_HEADS):
        s = aw[:, h * 16:(h + 1) * 16]
        m = jnp.max(s, axis=1, keepdims=True)
        e = jnp.exp(s - m)
        parts.append(e / jnp.sum(e, axis=1, keepdims=True))
    aws = jnp.concatenate(parts, axis=1)

    Wf = meta_ref[0:1, :]
    Hf = meta_ref[1:2, :]
    startf = meta_ref[2:3, :]
    hf = meta_ref[3:4, :]
    rpx = jnp.dot(rp_ref[:, 0:4], sel_ref[:], preferred_element_type=jnp.float32, precision=lax.Precision.HIGHEST)
    rpy = jnp.dot(rp_ref[:, 4:8], sel_ref[:], preferred_element_type=jnp.float32, precision=lax.Precision.HIGHEST)
    x = (rpx + sx / Wf) * Wf - 0.5
    y = (rpy + sy / Hf) * Hf - 0.5
    x0 = jnp.floor(x)
    y0 = jnp.floor(y)
    wx1 = x - x0
    wx0 = 1.0 - wx1
    wy1 = y - y0
    wy0 = 1.0 - wy1
    x1 = x0 + 1.0
    y1 = y0 + 1.0
    Wm1 = Wf - 1.0
    Hm1 = Hf - 1.0
    vx0 = ((x0 >= 0.0) & (x0 <= Wm1)).astype(jnp.float32)
    vx1 = ((x1 >= 0.0) & (x1 <= Wm1)).astype(jnp.float32)
    vy0 = ((y0 >= 0.0) & (y0 <= Hm1)).astype(jnp.float32)
    vy1 = ((y1 >= 0.0) & (y1 <= Hm1)).astype(jnp.float32)
    x0c = jnp.clip(x0, 0.0, Wm1)
    x1c = jnp.clip(x1, 0.0, Wm1)
    y0c = jnp.clip(y0, 0.0, Hm1)
    y1c = jnp.clip(y1, 0.0, Hm1)
    nb = (pl.program_id(0) // QT_PER_B).astype(jnp.float32) * float(LEN_IN * N_HEADS)

    def fidx(xc, yc):
        return (nb + (startf + yc * Wf + xc) * 8.0 + hf).astype(jnp.int32)

    wx0v = wx0 * vx0
    wx1v = wx1 * vx1
    wy0v = wy0 * vy0 * aws
    wy1v = wy1 * vy1 * aws
    idx_ref[:] = jnp.concatenate(
        [fidx(x0c, y0c), fidx(x1c, y0c), fidx(x0c, y1c), fidx(x1c, y1c)], axis=1)
    w_ref[:] = jnp.concatenate(
        [wx0v * wy0v, wx1v * wy0v, wx0v * wy1v, wx1v * wy1v], axis=1)


def _prep(q_flat, rp_flat, wso_p, bso_p, W_aw, b_aw, meta, sel):
    m = q_flat.shape[0]
    return pl.pallas_call(
        _prep_body,
        grid=(m // BQ,),
        in_specs=[pl.BlockSpec((BQ, D_MODEL), lambda i: (i, 0)),
                  pl.BlockSpec((BQ, 8), lambda i: (i, 0)),
                  pl.BlockSpec((D_MODEL, 2 * HLP), lambda i: (0, 0)),
                  pl.BlockSpec((1, 2 * HLP), lambda i: (0, 0)),
                  pl.BlockSpec((D_MODEL, HLP), lambda i: (0, 0)),
                  pl.BlockSpec((1, HLP), lambda i: (0, 0)),
                  pl.BlockSpec((8, HLP), lambda i: (0, 0)),
                  pl.BlockSpec((N_LEVELS, HLP), lambda i: (0, 0))],
        out_specs=[pl.BlockSpec((BQ, 4 * HLP), lambda i: (i, 0)),
                   pl.BlockSpec((BQ, 4 * HLP), lambda i: (i, 0))],
        out_shape=[jax.ShapeDtypeStruct((m, 4 * HLP), jnp.int32),
                   jax.ShapeDtypeStruct((m, 4 * HLP), jnp.float32)],
    )(q_flat, rp_flat, wso_p, bso_p.reshape(1, 2 * HLP), W_aw,
      b_aw.reshape(1, HLP), meta, sel)


NW = 32
G = 2
CH = G * 4


def _make_sc(nq):
    qpw = nq // NW
    nit = qpw // G

    def _sc_body(tab, idxh, wh, outh,
                 idx_v, w_v, rows_v, out_v, sem_g, sem_o):
        wid = lax.axis_index("s") * 2 + lax.axis_index("c")
        q0 = wid * qpw

        def stage(it, buf):
            qb = q0 + it * G
            pltpu.sync_copy(idxh.at[pl.ds(qb * 4, CH)], idx_v.at[buf])
            pltpu.sync_copy(wh.at[pl.ds(qb * 4, CH)], w_v.at[buf])
            for j in range(CH):
                pltpu.async_copy(tab.at[idx_v.at[buf, j]],
                                 rows_v.at[buf, pl.ds(j * 128, 128)],
                                 sem_g.at[buf])

        stage(0, 0)

        def it_body(it, carry):
            cur = lax.rem(it, 2)
            qb = q0 + it * G

            @pl.when(it + 1 < nit)
            def _():
                stage(it + 1, 1 - cur)

            for j in range(CH):
                pltpu.make_async_copy(tab.at[idx_v.at[cur, j]],
                                      rows_v.at[cur, pl.ds(j * 128, 128)],
                                      sem_g.at[cur]).wait()

            @pl.when(it >= 2)
            def _():
                pltpu.make_async_copy(out_v.at[cur],
                                      outh.at[pl.ds(qb, G)],
                                      sem_o.at[cur]).wait()

            for q in range(G):
                for h in range(N_HEADS):

                    def c_body(c, acc, q=q, h=h):
                        acc = list(acc)
                        wv = w_v[cur, q * 4 + c, pl.ds(h * 16, 16)]
                        ro = q * 512 + c * 128 + h * 16
                        for i in range(16):
                            wgt = wv[i]
                            k = (i & 3) * 2
                            acc[k] = acc[k] + wgt * rows_v[cur, ro + i, pl.ds(0, 16)]
                            acc[k + 1] = acc[k + 1] + wgt * rows_v[cur, ro + i, pl.ds(16, 16)]
                        return tuple(acc)

                    z = jnp.zeros((16,), jnp.float32)
                    acc = lax.fori_loop(0, 4, c_body, (z,) * 8)
                    a0 = (acc[0] + acc[2]) + (acc[4] + acc[6])
                    a1 = (acc[1] + acc[3]) + (acc[5] + acc[7])
                    out_v[cur, q, pl.ds(h * 32, 16)] = a0
                    out_v[cur, q, pl.ds(h * 32 + 16, 16)] = a1
            pltpu.async_copy(out_v.at[cur], outh.at[pl.ds(qb, G)],
                             sem_o.at[cur])
            return carry

        lax.fori_loop(0, nit, it_body, 0)
        for tail in (nit - 2, nit - 1):
            pltpu.make_async_copy(out_v.at[tail % 2],
                                  outh.at[pl.ds(q0 + tail * G, G)],
                                  sem_o.at[tail % 2]).wait()

    return functools.partial(
        pl.kernel,
        out_type=jax.ShapeDtypeStruct((nq, D_MODEL), jnp.float32),
        mesh=plsc.VectorSubcoreMesh(core_axis_name="c", subcore_axis_name="s"),
        scratch_types=[
            pltpu.VMEM((2, CH, 128), jnp.int32),
            pltpu.VMEM((2, CH, 128), jnp.float32),
            pltpu.VMEM((2, CH * 128, D_HEAD), jnp.float32),
            pltpu.VMEM((2, G, D_MODEL), jnp.float32),
            pltpu.SemaphoreType.DMA((2,)),
            pltpu.SemaphoreType.DMA((2,)),
        ],
        compiler_params=pltpu.CompilerParams(use_tc_tiling_on_sc=False),
    )(_sc_body)


_sc_gather = _make_sc(NQ)


def kernel(query, reference_points, input_flatten, input_spatial_shapes,
           input_level_start_index, W_so, b_so, W_aw, b_aw, W_v, b_v,
           W_out, b_out):
    del input_spatial_shapes, input_level_start_index
    q_flat = query.reshape(NQ, D_MODEL)
    rp_flat = reference_points.reshape(NQ, N_LEVELS, 2).transpose(0, 2, 1).reshape(NQ, 8)
    wso_p = W_so.reshape(D_MODEL, N_HEADS, N_LEVELS, N_POINTS, 2) \
        .transpose(0, 4, 1, 2, 3).reshape(D_MODEL, 2 * HLP)
    bso_p = b_so.reshape(N_HEADS, N_LEVELS, N_POINTS, 2) \
        .transpose(3, 0, 1, 2).reshape(2 * HLP)
    meta = jnp.asarray(_META_NP)
    sel = jnp.asarray(_SEL_NP)

    vtab = _matmul(input_flatten.reshape(NQ, D_MODEL), W_v, b_v, BR)
    vtab = vtab.reshape(NROWS, D_HEAD)
    idx, w = _prep(q_flat, rp_flat, wso_p, bso_p, W_aw, b_aw, meta, sel)
    out_sc = _sc_gather(vtab, idx.reshape(NQ * 4, HLP), w.reshape(NQ * 4, HLP))
    out = _matmul(out_sc, W_out, b_out, BR)
    return out.reshape(NB, LEN_Q, D_MODEL)
